# baseline (speedup 1.0000x reference)
import functools

import jax
import jax.numpy as jnp
from jax import lax
from jax.experimental import pallas as pl
from jax.experimental.pallas import tpu as pltpu
from jax.experimental.pallas import tpu_sc as plsc

D = 128
NW = 32
CHUNK = 64
K = 4


def _embed_lookup(idx, table):
    mesh = plsc.VectorSubcoreMesh(core_axis_name="c", subcore_axis_name="s")
    B = idx.shape[0]
    b_per_w = B // NW
    n_chunks = b_per_w // CHUNK
    n_groups = n_chunks // K
    assert n_chunks % K == 0 and n_groups >= 3

    @functools.partial(
        pl.kernel,
        mesh=mesh,
        out_type=jax.ShapeDtypeStruct((B, D), jnp.float32),
        scratch_types=[
            pltpu.VMEM((b_per_w,), jnp.int32),
            pltpu.VMEM((2, K, CHUNK, D), jnp.float32),
            pltpu.SemaphoreType.DMA,
            pltpu.SemaphoreType.DMA,
            pltpu.SemaphoreType.DMA,
        ],
    )
    def body(idx_hbm, table_hbm, out_hbm, idx_v, rows_v, gsem, osem0, osem1):
        wid = lax.axis_index("s") * 2 + lax.axis_index("c")
        base = wid * b_per_w
        pltpu.sync_copy(idx_hbm.at[pl.ds(base, b_per_w)], idx_v)
        osems = (osem0, osem1)

        def run_group(g, p, drain_prev):
            if drain_prev:
                for b in range(K):
                    pltpu.make_async_copy(
                        rows_v.at[p, b], out_hbm.at[pl.ds(base, CHUNK)],
                        osems[p],
                    ).wait()
            descs = []
            for b in range(K):
                off = (g * K + b) * CHUNK
                descs.append(pltpu.async_copy(
                    table_hbm.at[idx_v.at[pl.ds(off, CHUNK)]],
                    rows_v.at[p, b], gsem,
                ))
            for d in descs:
                d.wait()
            for b in range(K):
                off = (g * K + b) * CHUNK
                pltpu.async_copy(
                    rows_v.at[p, b], out_hbm.at[pl.ds(base + off, CHUNK)],
                    osems[p],
                )

        run_group(0, 0, False)
        run_group(1, 1, False)
        n_super = (n_groups - 2) // 2

        def super_body(s, carry):
            run_group(2 + 2 * s, 0, True)
            run_group(3 + 2 * s, 1, True)
            return carry

        lax.fori_loop(0, n_super, super_body, 0)
        if (n_groups - 2) % 2:
            run_group(n_groups - 1, 0, True)
        for p in range(2):
            for b in range(K):
                pltpu.make_async_copy(
                    rows_v.at[p, b], out_hbm.at[pl.ds(base, CHUNK)], osems[p],
                ).wait()

    return body(idx, table)


def kernel(x, table):
    n_batch, t = x.shape
    idx = x.T.reshape(n_batch * t).astype(jnp.int32)
    out = _embed_lookup(idx, table)
    return out.reshape(t, n_batch, D).transpose(1, 0, 2)


# speedup vs baseline: 1.0130x; 1.0130x over previous
import functools

import jax
import jax.numpy as jnp
from jax import lax
from jax.experimental import pallas as pl
from jax.experimental.pallas import tpu as pltpu
from jax.experimental.pallas import tpu_sc as plsc

D = 128
NW = 32
CHUNK = 80
K = 4


def _embed_lookup(idx, table):
    mesh = plsc.VectorSubcoreMesh(core_axis_name="c", subcore_axis_name="s")
    B = idx.shape[0]
    b_per_w = B // NW
    n_chunks = b_per_w // CHUNK
    n_groups = n_chunks // K
    assert n_chunks % K == 0 and n_groups >= 3

    @functools.partial(
        pl.kernel,
        mesh=mesh,
        out_type=jax.ShapeDtypeStruct((B, D), jnp.float32),
        scratch_types=[
            pltpu.VMEM((b_per_w,), jnp.int32),
            pltpu.VMEM((2, K, CHUNK, D), jnp.float32),
            pltpu.SemaphoreType.DMA,
            pltpu.SemaphoreType.DMA,
            pltpu.SemaphoreType.DMA,
        ],
    )
    def body(idx_hbm, table_hbm, out_hbm, idx_v, rows_v, gsem, osem0, osem1):
        wid = lax.axis_index("s") * 2 + lax.axis_index("c")
        base = wid * b_per_w
        pltpu.sync_copy(idx_hbm.at[pl.ds(base, b_per_w)], idx_v)
        osems = (osem0, osem1)

        def run_group(g, p, drain_prev):
            if drain_prev:
                for b in range(K):
                    pltpu.make_async_copy(
                        rows_v.at[p, b], out_hbm.at[pl.ds(base, CHUNK)],
                        osems[p],
                    ).wait()
            descs = []
            for b in range(K):
                off = (g * K + b) * CHUNK
                descs.append(pltpu.async_copy(
                    table_hbm.at[idx_v.at[pl.ds(off, CHUNK)]],
                    rows_v.at[p, b], gsem,
                ))
            for d in descs:
                d.wait()
            for b in range(K):
                off = (g * K + b) * CHUNK
                pltpu.async_copy(
                    rows_v.at[p, b], out_hbm.at[pl.ds(base + off, CHUNK)],
                    osems[p],
                )

        run_group(0, 0, False)
        run_group(1, 1, False)
        n_super = (n_groups - 2) // 2

        def super_body(s, carry):
            run_group(2 + 2 * s, 0, True)
            run_group(3 + 2 * s, 1, True)
            return carry

        lax.fori_loop(0, n_super, super_body, 0)
        if (n_groups - 2) % 2:
            run_group(n_groups - 1, 0, True)
        for p in range(2):
            for b in range(K):
                pltpu.make_async_copy(
                    rows_v.at[p, b], out_hbm.at[pl.ds(base, CHUNK)], osems[p],
                ).wait()

    return body(idx, table)


def kernel(x, table):
    n_batch, t = x.shape
    idx = x.T.reshape(n_batch * t).astype(jnp.int32)
    out = _embed_lookup(idx, table)
    return out.reshape(t, n_batch, D).transpose(1, 0, 2)
